# baseline (speedup 1.0000x reference)
import functools

import jax
import jax.numpy as jnp
from jax import lax
from jax.experimental import pallas as pl
from jax.experimental.pallas import tpu as pltpu
from jax.experimental.pallas import tpu_sc as plsc

VOCAB = 1000000
EMBED_DIM = 32
BATCH = 16384
HIST = 50

NC = 2
NS = 16
NW = NC * NS
NUNIT = HIST * (BATCH // 128)
PER_W = NUNIT // NW


def _mesh():
    return plsc.VectorSubcoreMesh(core_axis_name="c", subcore_axis_name="s")


NBLK_FULL = VOCAB // 128
BASE_BLK = NBLK_FULL // NW
TAIL_C0 = NBLK_FULL * 128


@functools.partial(
    pl.kernel,
    mesh=_mesh(),
    compiler_params=pltpu.CompilerParams(
        use_tc_tiling_on_sc=True, needs_layout_passes=False
    ),
    out_type=(
        jax.ShapeDtypeStruct((VOCAB * EMBED_DIM,), jnp.float32),
        jax.ShapeDtypeStruct((HIST * BATCH,), jnp.int32),
    ),
    scratch_types=[
        [pltpu.VMEM((EMBED_DIM, 128), jnp.float32)] * 2,
        [pltpu.VMEM((4096,), jnp.float32)] * 2,
        pltpu.VMEM((EMBED_DIM, 64), jnp.float32),
        pltpu.VMEM((2048,), jnp.float32),
        pltpu.VMEM((BATCH,), jnp.int32),
        [pltpu.SemaphoreType.DMA] * 2,
        [pltpu.SemaphoreType.DMA] * 2,
    ],
)
def _prep_kernel(xt_hbm, tt_hbm, tab_out, idx_out, fbuf, tbuf, fbuf64,
                 tbuf64, ibuf, isems, osems):
    wid = lax.axis_index("s") * NC + lax.axis_index("c")
    i16 = jnp.arange(16, dtype=jnp.int32)
    v32 = i16 * EMBED_DIM

    pltpu.sync_copy(xt_hbm.at[wid], ibuf)
    pltpu.sync_copy(ibuf, idx_out.at[pl.ds(wid * BATCH, BATCH)])

    @pl.when(wid < HIST - NW)
    def _():
        pltpu.sync_copy(xt_hbm.at[wid + NW], ibuf)
        pltpu.sync_copy(ibuf, idx_out.at[pl.ds((wid + NW) * BATCH, BATCH)])

    nfull = BASE_BLK + (wid < 4)

    def blk(i):
        return wid + NW * i

    def fire_in(b, i):
        pltpu.async_copy(
            tt_hbm.at[:, pl.ds(blk(i) * 128, 128)], fbuf[b], isems[b]
        )

    def drain_in(b):
        pltpu.make_async_copy(
            tt_hbm.at[:, pl.ds(0, 128)], fbuf[b], isems[b]
        ).wait()

    def transpose_blk(b):
        for d in range(EMBED_DIM):
            for q in range(8):
                vals = fbuf[b][d, pl.ds(q * 16, 16)]
                plsc.store_scatter(tbuf[b], [v32 + (q * 16 * EMBED_DIM + d)],
                                   vals)

    def store_out(b, i):
        pltpu.async_copy(
            tbuf[b], tab_out.at[pl.ds(blk(i) * 4096, 4096)], osems[b]
        )

    def drain_out(b):
        pltpu.make_async_copy(
            tbuf[b], tab_out.at[pl.ds(0, 4096)], osems[b]
        ).wait()

    fire_in(0, 0)

    def body(k, carry):
        for p in range(2):
            i = 2 * k + p
            nxt = 1 - p
            drain_in(p)

            @pl.when(i + 1 < nfull)
            def _():
                fire_in(nxt, i + 1)

            @pl.when(i >= 2)
            def _():
                drain_out(p)

            transpose_blk(p)
            store_out(p, i)
        return carry

    lax.fori_loop(0, BASE_BLK // 2, body, 0)

    @pl.when(wid < 4)
    def _():
        drain_in(0)
        drain_out(0)
        transpose_blk(0)
        store_out(0, BASE_BLK)

    @pl.when(wid == 4)
    def _():
        pltpu.sync_copy(tt_hbm.at[:, pl.ds(TAIL_C0, 64)], fbuf64)
        for d in range(EMBED_DIM):
            for q in range(4):
                vals = fbuf64[d, pl.ds(q * 16, 16)]
                plsc.store_scatter(tbuf64, [v32 + (q * 16 * EMBED_DIM + d)],
                                   vals)
        pltpu.sync_copy(
            tbuf64, tab_out.at[pl.ds(TAIL_C0 * EMBED_DIM, 2048)]
        )

    drain_out(0)
    drain_out(1)


@functools.partial(
    pl.kernel,
    mesh=_mesh(),
    compiler_params=pltpu.CompilerParams(
        use_tc_tiling_on_sc=False, needs_layout_passes=False
    ),
    out_type=jax.ShapeDtypeStruct((HIST * EMBED_DIM * BATCH,), jnp.float32),
    scratch_types=[
        [pltpu.VMEM((128,), jnp.int32)] * 2,
        [pltpu.VMEM((128, EMBED_DIM), jnp.float32)] * 2,
        [pltpu.VMEM((4096,), jnp.float32)] * 2,
        [pltpu.SemaphoreType.DMA] * 2,
        [pltpu.SemaphoreType.DMA] * 2,
        [pltpu.SemaphoreType.DMA] * 2,
    ],
)
def _gather_kernel(tab_hbm, idx_hbm, out_hbm, idx_v, rows_v, tbuf, isems,
                   gsems, osems):
    wid = lax.axis_index("s") * NC + lax.axis_index("c")
    i16 = jnp.arange(16, dtype=jnp.int32)
    a_lo = (i16 // 8) * 1024 + (i16 % 8) * 128
    a_hi = a_lo + 2048

    def unit_id(i):
        return wid + NW * i

    def fire_idx(b, i):
        pltpu.async_copy(idx_hbm.at[unit_id(i)], idx_v[b], isems[b])

    def wait_idx(b):
        pltpu.make_async_copy(idx_hbm.at[0], idx_v[b], isems[b]).wait()

    def fire_gather(b):
        pltpu.async_copy(tab_hbm.at[idx_v[b]], rows_v[b], gsems[b])

    def drain_gather(b):
        pltpu.make_async_copy(
            tab_hbm.at[pl.ds(0, 128)], rows_v[b], gsems[b]
        ).wait()

    def transpose(b):
        for cc in range(128):
            lo = rows_v[b][cc, pl.ds(0, 16)]
            hi = rows_v[b][cc, pl.ds(16, 16)]
            plsc.store_scatter(tbuf[b], [a_lo + cc], lo)
            plsc.store_scatter(tbuf[b], [a_hi + cc], hi)

    def store(b, i):
        u = unit_id(i)
        h = u // 128
        cc = lax.rem(u, 128)
        for r in range(4):
            off = ((h * 4 + r) * 128 + cc) * 1024
            pltpu.async_copy(
                tbuf[b].at[pl.ds(r * 1024, 1024)],
                out_hbm.at[pl.ds(off, 1024)],
                osems[b],
            )

    def drain_store(b):
        for _ in range(4):
            pltpu.make_async_copy(
                tbuf[b].at[pl.ds(0, 1024)], out_hbm.at[pl.ds(0, 1024)],
                osems[b],
            ).wait()

    fire_idx(0, 0)
    wait_idx(0)
    fire_gather(0)
    fire_idx(1, 1)

    def body(k, carry):
        for p in range(2):
            i = 2 * k + p
            nxt = 1 - p

            drain_gather(p)

            @pl.when(i + 1 < PER_W)
            def _():
                wait_idx(nxt)
                fire_gather(nxt)

            @pl.when(i + 2 < PER_W)
            def _():
                fire_idx(p, i + 2)

            @pl.when(i >= 2)
            def _():
                drain_store(p)

            transpose(p)
            store(p, i)
        return carry

    lax.fori_loop(0, PER_W // 2, body, 0)
    drain_store(0)
    drain_store(1)


def kernel(x, table):
    tab_lin, idx_lin = _prep_kernel(x.T, table.T)
    out_flat = _gather_kernel(
        tab_lin.reshape(VOCAB, EMBED_DIM), idx_lin.reshape(NUNIT, 128)
    )
    out5 = out_flat.reshape(HIST, 4, 128, 8, 128)
    return jnp.transpose(out5, (2, 4, 0, 1, 3)).reshape(BATCH, HIST, EMBED_DIM)


# speedup vs baseline: 1.1074x; 1.1074x over previous
import functools

import jax
import jax.numpy as jnp
from jax import lax
from jax.experimental import pallas as pl
from jax.experimental.pallas import tpu as pltpu
from jax.experimental.pallas import tpu_sc as plsc

VOCAB = 1000000
EMBED_DIM = 32
BATCH = 16384
HIST = 50

NC = 2
NS = 16
NW = NC * NS
NUNIT = HIST * (BATCH // 128)
PER_W = NUNIT // NW

NBLK_FULL = VOCAB // 128
BASE_BLK = NBLK_FULL // NW
TAIL_C0 = NBLK_FULL * 128
OUT_ROWS = HIST * 4 * 128 * 8


def _mesh():
    return plsc.VectorSubcoreMesh(core_axis_name="c", subcore_axis_name="s")


@functools.partial(
    pl.kernel,
    mesh=_mesh(),
    compiler_params=pltpu.CompilerParams(
        use_tc_tiling_on_sc=True, needs_layout_passes=False
    ),
    out_type=(
        jax.ShapeDtypeStruct((VOCAB * EMBED_DIM,), jnp.float32),
        jax.ShapeDtypeStruct((HIST * BATCH,), jnp.int32),
    ),
    scratch_types=[
        [pltpu.VMEM((EMBED_DIM, 129), jnp.float32)] * 2,
        [pltpu.VMEM((4096,), jnp.float32)] * 2,
        pltpu.VMEM((EMBED_DIM, 64), jnp.float32),
        pltpu.VMEM((2048,), jnp.float32),
        pltpu.VMEM((BATCH,), jnp.int32),
        [pltpu.SemaphoreType.DMA] * 2,
        [pltpu.SemaphoreType.DMA] * 2,
    ],
)
def _prep_kernel(xt_hbm, tt_hbm, tab_out, idx_out, fbuf, tbuf, fbuf64,
                 tbuf64, ibuf, isems, osems):
    wid = lax.axis_index("s") * NC + lax.axis_index("c")
    i16 = jnp.arange(16, dtype=jnp.int32)
    rsel = [i16, i16 + 16]

    pltpu.sync_copy(xt_hbm.at[wid], ibuf)
    pltpu.sync_copy(ibuf, idx_out.at[pl.ds(wid * BATCH, BATCH)])

    @pl.when(wid < HIST - NW)
    def _():
        pltpu.sync_copy(xt_hbm.at[wid + NW], ibuf)
        pltpu.sync_copy(ibuf, idx_out.at[pl.ds((wid + NW) * BATCH, BATCH)])

    nfull = BASE_BLK + (wid < 4)

    def blk(i):
        return wid + NW * i

    def fire_in(b, i):
        pltpu.async_copy(
            tt_hbm.at[:, pl.ds(blk(i) * 128, 128)],
            fbuf[b].at[:, pl.ds(0, 128)],
            isems[b],
        )

    def drain_in(b):
        pltpu.make_async_copy(
            tt_hbm.at[:, pl.ds(0, 128)], fbuf[b].at[:, pl.ds(0, 128)],
            isems[b],
        ).wait()

    def transpose_blk(b):
        for vl in range(128):
            col = jnp.full((16,), vl, dtype=jnp.int32)
            for h in range(2):
                v = plsc.load_gather(fbuf[b], [rsel[h], col])
                tbuf[b][pl.ds(vl * 32 + h * 16, 16)] = v

    def store_out(b, i):
        pltpu.async_copy(
            tbuf[b], tab_out.at[pl.ds(blk(i) * 4096, 4096)], osems[b]
        )

    def drain_out(b):
        pltpu.make_async_copy(
            tbuf[b], tab_out.at[pl.ds(0, 4096)], osems[b]
        ).wait()

    fire_in(0, 0)

    def body(k, carry):
        for p in range(2):
            i = 2 * k + p
            nxt = 1 - p
            drain_in(p)

            @pl.when(i + 1 < nfull)
            def _():
                fire_in(nxt, i + 1)

            @pl.when(i >= 2)
            def _():
                drain_out(p)

            transpose_blk(p)
            store_out(p, i)
        return carry

    lax.fori_loop(0, BASE_BLK // 2, body, 0)

    @pl.when(wid < 4)
    def _():
        drain_in(0)
        drain_out(0)
        transpose_blk(0)
        store_out(0, BASE_BLK)

    @pl.when(wid == 4)
    def _():
        pltpu.sync_copy(tt_hbm.at[:, pl.ds(TAIL_C0, 64)], fbuf64)
        v64 = i16 * EMBED_DIM
        for d in range(EMBED_DIM):
            for q in range(4):
                vals = fbuf64[d, pl.ds(q * 16, 16)]
                plsc.store_scatter(tbuf64,
                                   [v64 + (q * 16 * EMBED_DIM + d)], vals)
        pltpu.sync_copy(
            tbuf64, tab_out.at[pl.ds(TAIL_C0 * EMBED_DIM, 2048)]
        )

    drain_out(0)
    drain_out(1)


@functools.partial(
    pl.kernel,
    mesh=_mesh(),
    compiler_params=pltpu.CompilerParams(
        use_tc_tiling_on_sc=False, needs_layout_passes=False
    ),
    out_type=jax.ShapeDtypeStruct((OUT_ROWS, 128), jnp.float32),
    scratch_types=[
        [pltpu.VMEM((128,), jnp.int32)] * 2,
        [pltpu.VMEM((128, EMBED_DIM), jnp.float32)] * 2,
        [pltpu.VMEM((EMBED_DIM, 129), jnp.float32)] * 2,
        [pltpu.SemaphoreType.DMA] * 2,
        [pltpu.SemaphoreType.DMA] * 2,
        [pltpu.SemaphoreType.DMA] * 2,
    ],
)
def _gather_kernel(tab_hbm, idx_hbm, out_hbm, idx_v, rows_v, tbuf, isems,
                   gsems, osems):
    wid = lax.axis_index("s") * NC + lax.axis_index("c")
    i16 = jnp.arange(16, dtype=jnp.int32)
    rsel = [i16, i16 + 16]

    def unit_id(i):
        return wid + NW * i

    def fire_idx(b, i):
        pltpu.async_copy(idx_hbm.at[unit_id(i)], idx_v[b], isems[b])

    def wait_idx(b):
        pltpu.make_async_copy(idx_hbm.at[0], idx_v[b], isems[b]).wait()

    def fire_gather(b):
        pltpu.async_copy(tab_hbm.at[idx_v[b]], rows_v[b], gsems[b])

    def drain_gather(b):
        pltpu.make_async_copy(
            tab_hbm.at[pl.ds(0, 128)], rows_v[b], gsems[b]
        ).wait()

    def transpose(b):
        for cc in range(128):
            col = jnp.full((16,), cc, dtype=jnp.int32)
            lo = rows_v[b][cc, pl.ds(0, 16)]
            hi = rows_v[b][cc, pl.ds(16, 16)]
            plsc.store_scatter(tbuf[b], [rsel[0], col], lo)
            plsc.store_scatter(tbuf[b], [rsel[1], col], hi)

    def store(b, i):
        u = unit_id(i)
        h = u // 128
        cc = lax.rem(u, 128)
        for r in range(4):
            row0 = ((h * 4 + r) * 128 + cc) * 8
            pltpu.async_copy(
                tbuf[b].at[pl.ds(r * 8, 8), pl.ds(0, 128)],
                out_hbm.at[pl.ds(row0, 8), :],
                osems[b],
            )

    def drain_store(b):
        for _ in range(4):
            pltpu.make_async_copy(
                tbuf[b].at[pl.ds(0, 8), pl.ds(0, 128)],
                out_hbm.at[pl.ds(0, 8), :],
                osems[b],
            ).wait()

    fire_idx(0, 0)
    wait_idx(0)
    fire_gather(0)
    fire_idx(1, 1)

    def body(k, carry):
        for p in range(2):
            i = 2 * k + p
            nxt = 1 - p

            drain_gather(p)

            @pl.when(i + 1 < PER_W)
            def _():
                wait_idx(nxt)
                fire_gather(nxt)

            @pl.when(i + 2 < PER_W)
            def _():
                fire_idx(p, i + 2)

            @pl.when(i >= 2)
            def _():
                drain_store(p)

            transpose(p)
            store(p, i)
        return carry

    lax.fori_loop(0, PER_W // 2, body, 0)
    drain_store(0)
    drain_store(1)


def kernel(x, table):
    tab_lin, idx_lin = _prep_kernel(x.T, table.T)
    out2d = _gather_kernel(
        tab_lin.reshape(VOCAB, EMBED_DIM), idx_lin.reshape(NUNIT, 128)
    )
    out5 = out2d.reshape(HIST, 4, 128, 8, 128)
    return jnp.transpose(out5, (2, 4, 0, 1, 3)).reshape(BATCH, HIST, EMBED_DIM)


# speedup vs baseline: 1.6181x; 1.4612x over previous
import functools

import jax
import jax.numpy as jnp
from jax import lax
from jax.experimental import pallas as pl
from jax.experimental.pallas import tpu as pltpu
from jax.experimental.pallas import tpu_sc as plsc

VOCAB = 1000000
EMBED_DIM = 32
BATCH = 16384
HIST = 50

NC = 2
NS = 16
NW = NC * NS
NUNIT = HIST * (BATCH // 128)
PER_W = NUNIT // NW

NBLK_FULL = VOCAB // 128
BASE_BLK = NBLK_FULL // NW
TAIL_C0 = NBLK_FULL * 128
OUT_ROWS = HIST * 4 * 128 * 8


def _mesh():
    return plsc.VectorSubcoreMesh(core_axis_name="c", subcore_axis_name="s")


@functools.partial(
    pl.kernel,
    mesh=_mesh(),
    compiler_params=pltpu.CompilerParams(
        use_tc_tiling_on_sc=True, needs_layout_passes=False
    ),
    out_type=jax.ShapeDtypeStruct((HIST * BATCH,), jnp.int32),
    scratch_types=[pltpu.VMEM((BATCH,), jnp.int32)],
)
def _prep_idx(xt_hbm, idx_out, ibuf):
    wid = lax.axis_index("s") * NC + lax.axis_index("c")
    pltpu.sync_copy(xt_hbm.at[wid], ibuf)
    pltpu.sync_copy(ibuf, idx_out.at[pl.ds(wid * BATCH, BATCH)])

    @pl.when(wid < HIST - NW)
    def _():
        pltpu.sync_copy(xt_hbm.at[wid + NW], ibuf)
        pltpu.sync_copy(ibuf, idx_out.at[pl.ds((wid + NW) * BATCH, BATCH)])


@functools.partial(
    pl.kernel,
    mesh=_mesh(),
    compiler_params=pltpu.CompilerParams(
        use_tc_tiling_on_sc=False, needs_layout_passes=False
    ),
    out_type=jax.ShapeDtypeStruct((OUT_ROWS, 128), jnp.float32),
    scratch_types=[
        [pltpu.VMEM((128,), jnp.int32)] * 2,
        [pltpu.VMEM((128, EMBED_DIM), jnp.float32)] * 2,
        [pltpu.VMEM((EMBED_DIM, 129), jnp.float32)] * 2,
        [pltpu.SemaphoreType.DMA] * 2,
        [pltpu.SemaphoreType.DMA] * 2,
        [pltpu.SemaphoreType.DMA] * 2,
    ],
)
def _gather_kernel(tab_hbm, idx_hbm, out_hbm, idx_v, rows_v, tbuf, isems,
                   gsems, osems):
    wid = lax.axis_index("s") * NC + lax.axis_index("c")
    i16 = jnp.arange(16, dtype=jnp.int32)
    rsel = [i16, i16 + 16]

    def unit_id(i):
        return wid + NW * i

    def fire_idx(b, i):
        pltpu.async_copy(idx_hbm.at[unit_id(i)], idx_v[b], isems[b])

    def wait_idx(b):
        pltpu.make_async_copy(idx_hbm.at[0], idx_v[b], isems[b]).wait()

    def fire_gather(b):
        pltpu.async_copy(tab_hbm.at[idx_v[b]], rows_v[b], gsems[b])

    def drain_gather(b):
        pltpu.make_async_copy(
            tab_hbm.at[pl.ds(0, 128)], rows_v[b], gsems[b]
        ).wait()

    def transpose(b):
        for cc in range(128):
            col = jnp.full((16,), cc, dtype=jnp.int32)
            lo = rows_v[b][cc, pl.ds(0, 16)]
            hi = rows_v[b][cc, pl.ds(16, 16)]
            plsc.store_scatter(tbuf[b], [rsel[0], col], lo)
            plsc.store_scatter(tbuf[b], [rsel[1], col], hi)

    def store(b, i):
        u = unit_id(i)
        h = u // 128
        cc = lax.rem(u, 128)
        for r in range(4):
            row0 = ((h * 4 + r) * 128 + cc) * 8
            pltpu.async_copy(
                tbuf[b].at[pl.ds(r * 8, 8), pl.ds(0, 128)],
                out_hbm.at[pl.ds(row0, 8), :],
                osems[b],
            )

    def drain_store(b):
        for _ in range(4):
            pltpu.make_async_copy(
                tbuf[b].at[pl.ds(0, 8), pl.ds(0, 128)],
                out_hbm.at[pl.ds(0, 8), :],
                osems[b],
            ).wait()

    fire_idx(0, 0)
    wait_idx(0)
    fire_gather(0)
    fire_idx(1, 1)

    def body(k, carry):
        for p in range(2):
            i = 2 * k + p
            nxt = 1 - p

            drain_gather(p)

            @pl.when(i + 1 < PER_W)
            def _():
                wait_idx(nxt)
                fire_gather(nxt)

            @pl.when(i + 2 < PER_W)
            def _():
                fire_idx(p, i + 2)

            @pl.when(i >= 2)
            def _():
                drain_store(p)

            transpose(p)
            store(p, i)
        return carry

    lax.fori_loop(0, PER_W // 2, body, 0)
    drain_store(0)
    drain_store(1)


def kernel(x, table):
    idx_lin = _prep_idx(x.T)
    out2d = _gather_kernel(table, idx_lin.reshape(NUNIT, 128))
    out5 = out2d.reshape(HIST, 4, 128, 8, 128)
    return jnp.transpose(out5, (2, 4, 0, 1, 3)).reshape(BATCH, HIST, EMBED_DIM)
